# Initial kernel scaffold; baseline (speedup 1.0000x reference)
import functools
import jax
import jax.numpy as jnp
from jax import lax
from jax.experimental import pallas as pl
from jax.experimental.pallas import tpu as pltpu
from jax.experimental.pallas import tpu_sc as plsc

B, M, N, F = 4, 16384, 32768, 256
PAD = 8
MP = M + PAD
NSUB = 16
WPB = 8
JW = N // WPB
NCH1 = JW // 16
CH = 128
NCH3 = JW // CH

_mesh = plsc.VectorSubcoreMesh(core_axis_name="c", subcore_axis_name="s")


@functools.partial(
    pl.kernel,
    out_type=jax.ShapeDtypeStruct((B, N, F), jnp.float32),
    mesh=_mesh,
    scratch_types=[
        pltpu.VMEM((JW,), jnp.int32),
        pltpu.VMEM((N,), jnp.int32),
        pltpu.VMEM((M,), jnp.int32),
        pltpu.VMEM((JW,), jnp.int32),
        pltpu.VMEM((JW,), jnp.int32),
        pltpu.VMEM_SHARED((NSUB, N), jnp.int32),
        pltpu.VMEM((CH, F), jnp.float32),
        pltpu.VMEM((CH, F), jnp.float32),
        pltpu.SemaphoreType.DMA,
        pltpu.SemaphoreType.DMA,
    ],
)
def _upsample_sc(feats_hbm, interp_hbm, up_hbm, neg1_hbm, out_hbm,
                 up_v, pos_v, interp_v, acc_v, tmp_v, shared,
                 buf0, buf1, sem0, sem1):
    c = lax.axis_index("c")
    s = lax.axis_index("s")
    b = 2 * c + s // WPB
    r = s % WPB
    base = r * JW

    pltpu.sync_copy(up_hbm.at[b, pl.ds(base, JW)], up_v)
    pltpu.sync_copy(interp_hbm.at[b], interp_v)
    pltpu.sync_copy(neg1_hbm, pos_v)

    ii = lax.iota(jnp.int32, 16)
    ii_next = jnp.minimum(ii + 1, 15)
    gd = lax.GatherDimensionNumbers(
        offset_dims=(), collapsed_slice_dims=(0,), start_index_map=(0,))

    def ph1(ci, carry):
        idx = up_v[pl.ds(ci * 16, 16)]
        nvec = base + ci * 16 + ii
        comb = (idx << 15) | nvec
        scomb = lax.sort(comb)
        idx_s = lax.shift_right_logical(scomb, 15)
        n_s = scomb & 0x7FFF
        nxt = lax.gather(idx_s, ii_next[:, None], gd, slice_sizes=(1,),
                         mode=lax.GatherScatterMode.PROMISE_IN_BOUNDS)
        is_last = (idx_s != nxt) | (ii == 15)
        plsc.store_scatter(pos_v, [idx_s], n_s, mask=is_last)
        return carry

    lax.fori_loop(0, NCH1, ph1, 0)

    pltpu.sync_copy(pos_v, shared.at[s])
    plsc.subcore_barrier()

    g0 = (s // WPB) * WPB
    pltpu.sync_copy(shared.at[g0, pl.ds(base, JW)], acc_v)
    for k in range(1, WPB):
        pltpu.sync_copy(shared.at[g0 + k, pl.ds(base, JW)], tmp_v)

        def mg(i, carry):
            sl = pl.ds(i * 16, 16)
            acc_v[sl] = jnp.maximum(acc_v[sl], tmp_v[sl])
            return carry

        lax.fori_loop(0, JW // 16, mg, 0)

    boff = b * MP

    def mp(i, carry):
        sl = pl.ds(i * 16, 16)
        pos = acc_v[sl]
        cidx = jnp.maximum(pos - M, 0)
        ival = plsc.load_gather(interp_v, [cidx])
        row = jnp.where(pos >= M, ival, pos)
        row = jnp.where(pos >= 0, row, M)
        acc_v[sl] = row + boff
        return carry

    lax.fori_loop(0, JW // 16, mp, 0)

    bufs = (buf0, buf1)
    sems = (sem0, sem1)
    handles = [None, None]

    def start(k):
        idx_ref = acc_v.at[pl.ds(k * CH, CH)]
        return pltpu.async_copy(feats_hbm.at[idx_ref], bufs[k % 2], sems[k % 2])

    handles[0] = start(0)
    for k in range(NCH3):
        if k + 1 < NCH3:
            handles[(k + 1) % 2] = start(k + 1)
        handles[k % 2].wait()
        pltpu.sync_copy(bufs[k % 2], out_hbm.at[b, pl.ds(base + k * CH, CH)])


def kernel(feats, interpolate_idx, upsample_idx):
    assert feats.shape == (B, M, F) and upsample_idx.shape == (B, N)
    feats_ext = jnp.pad(feats, ((0, 0), (0, PAD), (0, 0))).reshape(B * MP, F)
    neg1 = jnp.full((N,), -1, jnp.int32)
    return _upsample_sc(feats_ext, interpolate_idx.astype(jnp.int32),
                        upsample_idx.astype(jnp.int32), neg1)


# speedup vs baseline: 2.0476x; 2.0476x over previous
import functools
import jax
import jax.numpy as jnp
from jax import lax
from jax.experimental import pallas as pl
from jax.experimental.pallas import tpu as pltpu
from jax.experimental.pallas import tpu_sc as plsc

B, M, N, F = 4, 16384, 32768, 256
PAD = 8
MP = M + PAD
NSUB = 16
WPB = 8
JW = N // WPB
NCH1 = JW // 16
CH = 64
NCH3 = JW // CH

_mesh = plsc.VectorSubcoreMesh(core_axis_name="c", subcore_axis_name="s")


@functools.partial(
    pl.kernel,
    out_type=jax.ShapeDtypeStruct((B, N, F), jnp.float32),
    mesh=_mesh,
    compiler_params=pltpu.CompilerParams(needs_layout_passes=False),
    scratch_types=[
        pltpu.VMEM((JW,), jnp.int32),
        pltpu.VMEM((N,), jnp.int32),
        pltpu.VMEM((M,), jnp.int32),
        pltpu.VMEM((JW,), jnp.int32),
        pltpu.VMEM((JW,), jnp.int32),
        pltpu.VMEM_SHARED((NSUB, N), jnp.int32),
        pltpu.VMEM((32,), jnp.int32),
        pltpu.VMEM((CH, F), jnp.float32),
        pltpu.VMEM((CH, F), jnp.float32),
        pltpu.SemaphoreType.DMA,
        pltpu.SemaphoreType.DMA,
    ],
)
def _upsample_sc(feats_hbm, interp_hbm, up_hbm, neg1_hbm, out_hbm,
                 up_v, pos_v, interp_v, acc_v, tmp_v, shared,
                 nbr_v, buf0, buf1, sem0, sem1):
    c = lax.axis_index("c")
    s = lax.axis_index("s")
    b = 2 * c + s // WPB
    r = s % WPB
    base = r * JW

    pltpu.sync_copy(up_hbm.at[b, pl.ds(base, JW)], up_v)
    pltpu.sync_copy(interp_hbm.at[b], interp_v)
    pltpu.sync_copy(neg1_hbm, pos_v)

    ii = lax.iota(jnp.int32, 16)
    nbr_v[pl.ds(16, 16)] = jnp.full((16,), -1, jnp.int32)

    def ph1(ci, carry):
        idx = up_v[pl.ds(ci * 16, 16)]
        nvec = base + ci * 16 + ii
        comb = (idx << 15) | nvec
        scomb, n_s = plsc.sort_key_val(comb, nvec)
        idx_s = lax.shift_right_logical(scomb, 15)
        nbr_v[pl.ds(0, 16)] = idx_s
        nxt = nbr_v[pl.ds(1, 16)]
        is_last = idx_s != nxt
        plsc.store_scatter(pos_v, [idx_s], n_s, mask=is_last)
        return carry

    lax.fori_loop(0, NCH1, ph1, 0)

    pltpu.sync_copy(pos_v, shared.at[s])
    plsc.subcore_barrier()

    g0 = (s // WPB) * WPB
    pltpu.sync_copy(shared.at[g0, pl.ds(base, JW)], acc_v)
    for k in range(1, WPB):
        pltpu.sync_copy(shared.at[g0 + k, pl.ds(base, JW)], tmp_v)

        def mg(i, carry):
            sl = pl.ds(i * 16, 16)
            acc_v[sl] = jnp.maximum(acc_v[sl], tmp_v[sl])
            return carry

        lax.fori_loop(0, JW // 16, mg, 0)

    boff = b * MP

    def mp(i, carry):
        sl = pl.ds(i * 16, 16)
        pos = acc_v[sl]
        cidx = jnp.maximum(pos - M, 0)
        ival = plsc.load_gather(interp_v, [cidx])
        row = jnp.where(pos >= M, ival, pos)
        row = jnp.where(pos >= 0, row, M)
        acc_v[sl] = row + boff
        return carry

    lax.fori_loop(0, JW // 16, mp, 0)

    bufs = (buf0, buf1)
    sems = (sem0, sem1)
    handles = [None, None]

    def start(k):
        idx_ref = acc_v.at[pl.ds(k * CH, CH)]
        return pltpu.async_copy(feats_hbm.at[idx_ref], bufs[k % 2], sems[k % 2])

    handles[0] = start(0)
    for k in range(NCH3):
        if k + 1 < NCH3:
            handles[(k + 1) % 2] = start(k + 1)
        handles[k % 2].wait()
        pltpu.sync_copy(bufs[k % 2], out_hbm.at[b, pl.ds(base + k * CH, CH)])


def kernel(feats, interpolate_idx, upsample_idx):
    assert feats.shape == (B, M, F) and upsample_idx.shape == (B, N)
    feats_ext = jnp.pad(feats, ((0, 0), (0, PAD), (0, 0))).reshape(B * MP, F)
    neg1 = jnp.full((N,), -1, jnp.int32)
    return _upsample_sc(feats_ext, interpolate_idx.astype(jnp.int32),
                        upsample_idx.astype(jnp.int32), neg1)


# speedup vs baseline: 8.6312x; 4.2153x over previous
import functools
import jax
import jax.numpy as jnp
from jax import lax
from jax.experimental import pallas as pl
from jax.experimental.pallas import tpu as pltpu
from jax.experimental.pallas import tpu_sc as plsc

B, M, N, F = 4, 16384, 32768, 256
PAD = 128
MP = M + PAD
NSUB = 16
WPB = 8
JW = N // WPB
NCH1 = JW // 16
CH = 32
DEPTH = 4
NCH3 = JW // CH

_mesh = plsc.VectorSubcoreMesh(core_axis_name="c", subcore_axis_name="s")


@functools.partial(
    pl.kernel,
    out_type=jax.ShapeDtypeStruct((B, N, F), jnp.float32),
    mesh=_mesh,
    compiler_params=pltpu.CompilerParams(needs_layout_passes=False),
    scratch_types=[
        pltpu.VMEM((JW,), jnp.int32),
        pltpu.VMEM((N,), jnp.int32),
        pltpu.VMEM((M,), jnp.int32),
        pltpu.VMEM((JW,), jnp.int32),
        pltpu.VMEM((JW,), jnp.int32),
        pltpu.VMEM_SHARED((NSUB, N), jnp.int32),
        pltpu.VMEM((32,), jnp.int32),
        [pltpu.VMEM((CH, F), jnp.float32)] * DEPTH,
        [pltpu.SemaphoreType.DMA] * DEPTH,
    ],
)
def _upsample_sc(feats_hbm, interp_hbm, up_hbm, neg1_hbm, out_hbm,
                 up_v, pos_v, interp_v, acc_v, tmp_v, shared,
                 nbr_v, bufs, sems):
    c = lax.axis_index("c")
    s = lax.axis_index("s")
    b = 2 * c + s // WPB
    r = s % WPB
    base = r * JW

    pltpu.sync_copy(up_hbm.at[b, pl.ds(base, JW)], up_v)
    pltpu.sync_copy(interp_hbm.at[b], interp_v)
    pltpu.sync_copy(neg1_hbm, pos_v)

    ii = lax.iota(jnp.int32, 16)
    nbr_v[pl.ds(16, 16)] = jnp.full((16,), -1, jnp.int32)

    def ph1(ci, carry):
        idx = up_v[pl.ds(ci * 16, 16)]
        nvec = base + ci * 16 + ii
        comb = (idx << 15) | nvec
        scomb, n_s = plsc.sort_key_val(comb, nvec)
        idx_s = lax.shift_right_logical(scomb, 15)
        nbr_v[pl.ds(0, 16)] = idx_s
        nxt = nbr_v[pl.ds(1, 16)]
        is_last = idx_s != nxt
        plsc.store_scatter(pos_v, [idx_s], n_s, mask=is_last)
        return carry

    lax.fori_loop(0, NCH1, ph1, 0)

    pltpu.sync_copy(pos_v, shared.at[s])
    plsc.subcore_barrier()

    g0 = (s // WPB) * WPB
    pltpu.sync_copy(shared.at[g0, pl.ds(base, JW)], acc_v)
    for k in range(1, WPB):
        pltpu.sync_copy(shared.at[g0 + k, pl.ds(base, JW)], tmp_v)

        def mg(i, carry):
            sl = pl.ds(i * 16, 16)
            acc_v[sl] = jnp.maximum(acc_v[sl], tmp_v[sl])
            return carry

        lax.fori_loop(0, JW // 16, mg, 0)

    boff = b * MP

    def mp(i, carry):
        sl = pl.ds(i * 16, 16)
        pos = acc_v[sl]
        cidx = jnp.maximum(pos - M, 0)
        ival = plsc.load_gather(interp_v, [cidx])
        row = jnp.where(pos >= M, ival, pos)
        zrow = M + ((i * 16 + ii) & (PAD - 1))
        row = jnp.where(pos >= 0, row, zrow)
        acc_v[sl] = row + boff
        return carry

    lax.fori_loop(0, JW // 16, mp, 0)

    handles = [None] * DEPTH

    def start(k):
        idx_ref = acc_v.at[pl.ds(k * CH, CH)]
        return pltpu.async_copy(feats_hbm.at[idx_ref],
                                bufs[k % DEPTH], sems[k % DEPTH])

    for k in range(DEPTH - 1):
        handles[k] = start(k)
    for k in range(NCH3):
        if k + DEPTH - 1 < NCH3:
            handles[(k + DEPTH - 1) % DEPTH] = start(k + DEPTH - 1)
        handles[k % DEPTH].wait()
        pltpu.sync_copy(bufs[k % DEPTH],
                        out_hbm.at[b, pl.ds(base + k * CH, CH)])


def kernel(feats, interpolate_idx, upsample_idx):
    assert feats.shape == (B, M, F) and upsample_idx.shape == (B, N)
    feats_ext = jnp.pad(feats, ((0, 0), (0, PAD), (0, 0))).reshape(B * MP, F)
    neg1 = jnp.full((N,), -1, jnp.int32)
    return _upsample_sc(feats_ext, interpolate_idx.astype(jnp.int32),
                        upsample_idx.astype(jnp.int32), neg1)
